# baseline (speedup 1.0000x reference)
import jax
import jax.numpy as jnp
from jax import lax
from jax.experimental import pallas as pl
from jax.experimental.pallas import tpu as pltpu
from jax.experimental.pallas import tpu_sc as plsc

_B = 16384
_C = 1000
_NW = 32
_RPW = _B // _NW
_CH = 64
_NCH = _RPW // _CH


def _sc_probe_body(x_hbm, out_hbm, buf, acc, out_v):
    wid = lax.axis_index("s") * 2 + lax.axis_index("c")
    base = wid * _RPW
    acc[...] = jnp.zeros((16,), jnp.float32)
    for c in range(_NCH):
        pltpu.sync_copy(x_hbm.at[pl.ds(base + c * _CH, _CH)], buf)
        acc[...] = acc[...] + buf[0, pl.ds(0, 16)]
    out_v[...] = acc[...]
    pltpu.sync_copy(out_v, out_hbm.at[wid])


@jax.jit
def kernel(input, target):
    probe = pl.kernel(
        _sc_probe_body,
        out_type=jax.ShapeDtypeStruct((_NW, 16), jnp.float32),
        mesh=plsc.VectorSubcoreMesh(
            core_axis_name="c", subcore_axis_name="s", num_cores=2, num_subcores=16
        ),
        scratch_types=[
            pltpu.VMEM((_CH, _C), jnp.float32),
            pltpu.VMEM((16,), jnp.float32),
            pltpu.VMEM((16,), jnp.float32),
        ],
    )(input)
    return jnp.sum(probe)


# speedup vs baseline: 1.0751x; 1.0751x over previous
import jax
import jax.numpy as jnp
from jax import lax
from jax.experimental import pallas as pl
from jax.experimental.pallas import tpu as pltpu

_B = 16384
_C = 1000
_BR = 2048
_K = _B // 2


def _loss_body(x_ref, t_ref, loss_ref):
    x = x_ref[...]
    t = t_ref[...]
    m = jnp.max(x, axis=1)
    s = jnp.sum(jnp.exp(x - m[:, None]), axis=1)
    lse = m + jnp.log(s)
    col = lax.broadcasted_iota(jnp.int32, x.shape, 1)
    tgt = jnp.sum(jnp.where(col == t[:, None], x, 0.0), axis=1)
    loss_ref[...] = lse - tgt


def _topk_sum_body(loss_ref, out_ref):
    x = loss_ref[...]
    bits = lax.bitcast_convert_type(x, jnp.int32)
    ukey = lax.bitcast_convert_type(
        jnp.where(bits < 0, ~bits, bits | jnp.int32(-2147483648)), jnp.uint32
    )

    def step(i, p):
        c = p | (jnp.uint32(1) << (jnp.uint32(31) - i.astype(jnp.uint32)))
        cnt = jnp.sum((ukey >= c).astype(jnp.int32))
        return jnp.where(cnt >= _K, c, p)

    p = lax.fori_loop(0, 32, step, jnp.uint32(0))
    pi = lax.bitcast_convert_type(p, jnp.int32)
    vbits = jnp.where(pi < 0, pi & jnp.int32(0x7FFFFFFF), ~pi)
    v = lax.bitcast_convert_type(vbits, jnp.float32)
    sel = ukey > p
    cnt_gt = jnp.sum(sel.astype(jnp.int32))
    s = jnp.sum(jnp.where(sel, x, 0.0))
    rem = (_K - cnt_gt).astype(jnp.float32)
    out_ref[0, 0] = s + jnp.where(cnt_gt == _K, 0.0, rem * v)


@jax.jit
def kernel(input, target):
    loss = pl.pallas_call(
        _loss_body,
        grid=(_B // _BR,),
        in_specs=[
            pl.BlockSpec((_BR, _C), lambda i: (i, 0)),
            pl.BlockSpec((_BR,), lambda i: (i,)),
        ],
        out_specs=pl.BlockSpec((_BR,), lambda i: (i,)),
        out_shape=jax.ShapeDtypeStruct((_B,), jnp.float32),
    )(input, target)

    out = pl.pallas_call(
        _topk_sum_body,
        out_shape=jax.ShapeDtypeStruct((1, 1), jnp.float32),
        out_specs=pl.BlockSpec(memory_space=pltpu.SMEM),
    )(loss.reshape(128, 128))
    return out[0, 0]
